# Initial kernel scaffold; baseline (speedup 1.0000x reference)
import jax
import jax.numpy as jnp
from jax.experimental import pallas as pl


def kernel(is_electron_logit, positions, position_std_dev_cholesky, true_segmap, binary_mask_logits, portion_logits, occupancy_logits, incidence_points, matched_pred, occupancy_target):
    raise NotImplementedError("write your pallas kernel here")


# speedup vs baseline: 2.4273x; 2.4273x over previous
import functools

import jax
import jax.numpy as jnp
from jax.experimental import pallas as pl
from jax.experimental.pallas import tpu as pltpu

B, Q, T, H, W = 4, 128, 64, 64, 64
HW = H * W
WIN = 7
HALF = WIN // 2
C_OCC = 8
NO_ELECTRON_WEIGHT = 0.1
LOG_2PI = 1.8378770664093453


def _bce(x, y):
    return jnp.maximum(x, 0.0) - x * y + jnp.log1p(jnp.exp(-jnp.abs(x)))


def _loss_kernel(portion_ref, binary_ref, true_ref, matched_ref, inc_ref,
                 ie_ref, packed_ref, occ_ref, occ_oh_ref, out_ref, acc_ref):
    b = pl.program_id(0)

    matched = matched_ref[0]
    q_iota = jax.lax.broadcasted_iota(jnp.int32, (Q, T), 0)
    onehot = (q_iota == matched).astype(jnp.float32)

    true_b = true_ref[0]

    rp = jax.lax.dot_general(
        portion_ref[0], onehot, (((1,), (0,)), ((), ())),
        precision=jax.lax.Precision.HIGHEST,
        preferred_element_type=jnp.float32)
    p = jax.nn.sigmoid(rp)
    num_t = 2.0 * jnp.sum(p * true_b, axis=0, keepdims=True)
    den_t = jnp.sum(p, axis=0, keepdims=True) + jnp.sum(true_b, axis=0, keepdims=True)
    dice_b = jnp.sum(1.0 - (num_t + 1.0) / (den_t + 1.0))

    rb = jax.lax.dot_general(
        binary_ref[0], onehot, (((1,), (0,)), ((), ())),
        precision=jax.lax.Precision.HIGHEST,
        preferred_element_type=jnp.float32)
    r_t = jnp.floor(inc_ref[0, 0:1, :]).astype(jnp.int32)
    c_t = jnp.floor(inc_ref[0, 1:2, :]).astype(jnp.int32)
    pix = jax.lax.broadcasted_iota(jnp.int32, (HW, T), 0)
    hh = pix // W
    ww = pix % W
    wr = jnp.zeros((HW, T), jnp.float32)
    wc = jnp.zeros((HW, T), jnp.float32)
    for d in range(-HALF, HALF + 1):
        wr = wr + (hh == jnp.clip(r_t + d, 0, H - 1)).astype(jnp.float32)
        wc = wc + (ww == jnp.clip(c_t + d, 0, W - 1)).astype(jnp.float32)
    bce_b = jnp.sum(wr * wc * _bce(rb, true_b))

    labels = jnp.max(onehot, axis=1, keepdims=True)
    wts = jnp.where(labels > 0.0, 1.0, NO_ELECTRON_WEIGHT)
    x_ie = ie_ref[0].reshape(Q, 1)
    class_b = jnp.sum(wts * _bce(x_ie, labels))

    g = jax.lax.dot_general(
        onehot, packed_ref[0], (((0,), (0,)), ((), ())),
        precision=jax.lax.Precision.HIGHEST,
        preferred_element_type=jnp.float32)
    ix = inc_ref[0, 0:1, :].reshape(T, 1)
    iy = inc_ref[0, 1:2, :].reshape(T, 1)
    d0 = ix - g[:, 0:1]
    d1 = iy - g[:, 1:2]
    l00 = g[:, 2:3]
    l10 = g[:, 3:4]
    l11 = g[:, 4:5]
    z0 = d0 / l00
    z1 = (d1 - l10 * z0) / l11
    nll_b = jnp.sum(0.5 * (z0 * z0 + z1 * z1)
                    + jnp.log(jnp.abs(l00)) + jnp.log(jnp.abs(l11)) + LOG_2PI)

    @pl.when(b == 0)
    def _init():
        for i in range(4):
            acc_ref[i] = 0.0

    acc_ref[0] = acc_ref[0] + class_b
    acc_ref[1] = acc_ref[1] + bce_b
    acc_ref[2] = acc_ref[2] + dice_b
    acc_ref[3] = acc_ref[3] + nll_b

    @pl.when(b == B - 1)
    def _final():
        xo = occ_ref[:, :]
        m = jnp.max(xo, axis=1, keepdims=True)
        lse = m + jnp.log(jnp.sum(jnp.exp(xo - m), axis=1, keepdims=True))
        occ_loss = -jnp.sum(occ_oh_ref[:, :] * (xo - lse)) / B
        out_ref[0] = (acc_ref[0] / (B * Q)
                      + acc_ref[1] / (B * T * WIN * WIN)
                      + acc_ref[2] / (B * T)
                      + acc_ref[3] / (B * T)
                      + occ_loss)


@jax.jit
def kernel(is_electron_logit, positions, position_std_dev_cholesky, true_segmap,
           binary_mask_logits, portion_logits, occupancy_logits, incidence_points,
           matched_pred, occupancy_target):
    portion = portion_logits.reshape(B, HW, Q)
    binary = binary_mask_logits.reshape(B, HW, Q)
    true = true_segmap.reshape(B, HW, T)
    matched3 = matched_pred.reshape(B, 1, T)
    inc_t = incidence_points.transpose(0, 2, 1)
    ie = is_electron_logit.reshape(B, 1, Q)
    pos = positions.reshape(B, Q, 2)
    chol = position_std_dev_cholesky.reshape(B, Q, 2, 2)
    packed = jnp.concatenate(
        [pos, chol[..., 0, 0:1], chol[..., 1, 0:1], chol[..., 1, 1:2],
         jnp.zeros((B, Q, 3), jnp.float32)], axis=-1)
    occ_oh = (occupancy_target[:, None] ==
              jnp.arange(C_OCC, dtype=jnp.int32)[None, :]).astype(jnp.float32)

    out = pl.pallas_call(
        _loss_kernel,
        grid=(B,),
        in_specs=[
            pl.BlockSpec((1, HW, Q), lambda b: (b, 0, 0)),
            pl.BlockSpec((1, HW, Q), lambda b: (b, 0, 0)),
            pl.BlockSpec((1, HW, T), lambda b: (b, 0, 0)),
            pl.BlockSpec((1, 1, T), lambda b: (b, 0, 0)),
            pl.BlockSpec((1, 2, T), lambda b: (b, 0, 0)),
            pl.BlockSpec((1, 1, Q), lambda b: (b, 0, 0)),
            pl.BlockSpec((1, Q, 8), lambda b: (b, 0, 0)),
            pl.BlockSpec((B, C_OCC), lambda b: (0, 0)),
            pl.BlockSpec((B, C_OCC), lambda b: (0, 0)),
        ],
        out_specs=pl.BlockSpec(memory_space=pltpu.SMEM),
        out_shape=jax.ShapeDtypeStruct((1,), jnp.float32),
        scratch_shapes=[pltpu.SMEM((8,), jnp.float32)],
    )(portion, binary, true, matched3, inc_t, ie, packed, occupancy_logits, occ_oh)
    return out[0]
